# Initial kernel scaffold; baseline (speedup 1.0000x reference)
import jax
import jax.numpy as jnp
from jax.experimental import pallas as pl


def kernel(x, ei, ew, W_in, b_in, W1, b1, a1s, a1d, W2, b2, a2s, a2d, W_out, b_out):
    raise NotImplementedError("write your pallas kernel here")


# speedup vs baseline: 32.7500x; 32.7500x over previous
import functools

import jax
import jax.numpy as jnp
from jax import lax
from jax.experimental import pallas as pl
from jax.experimental.pallas import tpu as pltpu
from jax.experimental.pallas import tpu_sc as plsc

N = 10000
NPAD = 10240
D_IN = 128
H = 64
D_OUT = 64
E = 320000
ETOT = E + N

NC = 2
NS = 16
NW = NC * NS
CHUNK = 128
CHUNKS = 81
TPW = CHUNK * CHUNKS
EPAD = NW * TPW
ROWS_PER_TILE = NPAD // NS

RB = 1024
NB = NPAD // RB

_NEG = 0.2


def _leaky(v):
    return jnp.where(v >= 0, v, _NEG * v)


def _proj_body(h_ref, W_ref, b_ref, as_ref, ad_ref,
               z_ref, s_ref, d_ref, m_ref, acc_ref):
    i = pl.program_id(0)
    z = jnp.dot(h_ref[...], W_ref[...], preferred_element_type=jnp.float32)
    z = z + b_ref[...]
    z_ref[...] = z
    s = jnp.dot(z, as_ref[...], preferred_element_type=jnp.float32)
    d = jnp.dot(z, ad_ref[...], preferred_element_type=jnp.float32)
    s_ref[...] = s
    d_ref[...] = d
    bs, bd = jnp.max(s), jnp.max(d)
    ps = jnp.where(i == 0, -jnp.inf, acc_ref[0])
    pd = jnp.where(i == 0, -jnp.inf, acc_ref[1])
    acc_ref[0] = jnp.maximum(ps, bs)
    acc_ref[1] = jnp.maximum(pd, bd)

    @pl.when(i == NB - 1)
    def _():
        m_ref[...] = jnp.full((1, 1), _leaky(acc_ref[0] + acc_ref[1]),
                              jnp.float32)


def _proj(h, W, b, a_s, a_d):
    k = h.shape[1]
    return pl.pallas_call(
        _proj_body,
        grid=(NB,),
        in_specs=[
            pl.BlockSpec((RB, k), lambda i: (i, 0)),
            pl.BlockSpec((k, H), lambda i: (0, 0)),
            pl.BlockSpec((1, H), lambda i: (0, 0)),
            pl.BlockSpec((H, 1), lambda i: (0, 0)),
            pl.BlockSpec((H, 1), lambda i: (0, 0)),
        ],
        out_specs=[
            pl.BlockSpec((RB, H), lambda i: (i, 0)),
            pl.BlockSpec((RB, 1), lambda i: (i, 0)),
            pl.BlockSpec((RB, 1), lambda i: (i, 0)),
            pl.BlockSpec((1, 1), lambda i: (0, 0)),
        ],
        out_shape=[
            jax.ShapeDtypeStruct((NPAD, H), jnp.float32),
            jax.ShapeDtypeStruct((NPAD, 1), jnp.float32),
            jax.ShapeDtypeStruct((NPAD, 1), jnp.float32),
            jax.ShapeDtypeStruct((1, 1), jnp.float32),
        ],
        scratch_shapes=[pltpu.SMEM((2,), jnp.float32)],
    )(h, W, b.reshape(1, H), a_s.reshape(H, 1), a_d.reshape(H, 1))


def _comb_body(num_ref, den_ref, x_ref):
    nsum = num_ref[0] + num_ref[1]
    dsum = den_ref[0, :] + den_ref[1, :]
    x_ref[...] = nsum / (dsum[:, None] + 1e-16)


def _combine(num, den):
    return pl.pallas_call(
        _comb_body,
        grid=(NB,),
        in_specs=[
            pl.BlockSpec((NC, RB, H), lambda i: (0, i, 0)),
            pl.BlockSpec((2, RB), lambda i: (0, i)),
        ],
        out_specs=pl.BlockSpec((RB, H), lambda i: (i, 0)),
        out_shape=jax.ShapeDtypeStruct((NPAD, H), jnp.float32),
    )(num, den)


def _inproj_body(x_ref, W_ref, b_ref, o_ref):
    o_ref[...] = jnp.dot(x_ref[...], W_ref[...],
                         preferred_element_type=jnp.float32) + b_ref[...]


def _inproj(x, W, b):
    return pl.pallas_call(
        _inproj_body,
        grid=(NB,),
        in_specs=[
            pl.BlockSpec((RB, D_IN), lambda i: (i, 0)),
            pl.BlockSpec((D_IN, H), lambda i: (0, 0)),
            pl.BlockSpec((1, H), lambda i: (0, 0)),
        ],
        out_specs=pl.BlockSpec((RB, H), lambda i: (i, 0)),
        out_shape=jax.ShapeDtypeStruct((NPAD, H), jnp.float32),
    )(x, W, b.reshape(1, H))


def _outproj_body(x1_ref, x2_ref, x3_ref, W_ref, b_ref, o_ref):
    Wo = W_ref[...]
    acc = jnp.dot(x1_ref[...], Wo[0:H], preferred_element_type=jnp.float32)
    acc += jnp.dot(x2_ref[...], Wo[H:2 * H], preferred_element_type=jnp.float32)
    acc += jnp.dot(x3_ref[...], Wo[2 * H:], preferred_element_type=jnp.float32)
    o_ref[...] = acc + b_ref[...]


def _outproj(x1, x2, x3, W, b):
    return pl.pallas_call(
        _outproj_body,
        grid=(NB,),
        in_specs=[
            pl.BlockSpec((RB, H), lambda i: (i, 0)),
            pl.BlockSpec((RB, H), lambda i: (i, 0)),
            pl.BlockSpec((RB, H), lambda i: (i, 0)),
            pl.BlockSpec((3 * H, D_OUT), lambda i: (0, 0)),
            pl.BlockSpec((1, D_OUT), lambda i: (0, 0)),
        ],
        out_specs=pl.BlockSpec((RB, D_OUT), lambda i: (i, 0)),
        out_shape=jax.ShapeDtypeStruct((NPAD, D_OUT), jnp.float32),
    )(x1, x2, x3, W, b.reshape(1, D_OUT))


def _sc_edge_body(z_hbm, s_hbm, d_hbm, m_hbm, src_hbm, dst_hbm, w_hbm,
                  num_out, den_out,
                  s_tab, d_tab, m_v, src_t, dst_t, w_t, ex_c, zrows,
                  zerob, zerod, num_acc, den_acc, sem):
    c = lax.axis_index("c")
    sid = lax.axis_index("s")
    wid = sid * NC + c

    def _zb(i, _):
        for q in range(4):
            zerob[i, pl.ds(q * 16, 16)] = jnp.zeros((16,), jnp.float32)
        return 0
    lax.fori_loop(0, 64, _zb, 0)

    def _zd(i, _):
        zerod[pl.ds(i * 16, 16)] = jnp.zeros((16,), jnp.float32)
        return 0
    lax.fori_loop(0, ROWS_PER_TILE // 16, _zd, 0)

    for r in range(ROWS_PER_TILE // 64):
        pltpu.sync_copy(zerob, num_acc.at[pl.ds(sid * ROWS_PER_TILE + r * 64, 64)])
    pltpu.sync_copy(zerod, den_acc.at[pl.ds(sid * ROWS_PER_TILE, ROWS_PER_TILE)])

    pltpu.sync_copy(s_hbm, s_tab)
    pltpu.sync_copy(d_hbm, d_tab)
    pltpu.sync_copy(m_hbm, m_v)
    pltpu.sync_copy(src_hbm.at[wid], src_t)
    pltpu.sync_copy(dst_hbm.at[wid], dst_t)
    pltpu.sync_copy(w_hbm.at[wid], w_t)
    plsc.subcore_barrier()

    mv = m_v[...]

    def _chunk(ci, _):
        pltpu.async_copy(z_hbm.at[src_t.at[ci]], zrows, sem).wait()

        for j in range(CHUNK // 16):
            sj = src_t[ci, pl.ds(j * 16, 16)]
            dj = dst_t[ci, pl.ds(j * 16, 16)]
            sv = plsc.load_gather(s_tab, [sj])
            dv = plsc.load_gather(d_tab, [dj])
            e = _leaky(sv + dv)
            ex = jnp.exp(e - mv) * w_t[ci, pl.ds(j * 16, 16)]
            ex_c[pl.ds(j * 16, 16)] = ex

        def _row(i, _):
            exi = plsc.load_gather(ex_c, [jnp.full((16,), i, jnp.int32)])
            for q in range(4):
                zrows[i, pl.ds(q * 16, 16)] = zrows[i, pl.ds(q * 16, 16)] * exi
            return 0
        lax.fori_loop(0, CHUNK, _row, 0)

        pltpu.sync_copy(zrows, num_acc.at[dst_t.at[ci]], add=True)
        pltpu.sync_copy(ex_c, den_acc.at[dst_t.at[ci]], add=True)
        return 0

    lax.fori_loop(0, CHUNKS, _chunk, 0)
    plsc.subcore_barrier()

    row0 = sid * ROWS_PER_TILE
    pltpu.sync_copy(num_acc.at[pl.ds(row0, ROWS_PER_TILE)],
                    num_out.at[c, pl.ds(row0, ROWS_PER_TILE)])
    pltpu.sync_copy(den_acc.at[pl.ds(row0, ROWS_PER_TILE)],
                    den_out.at[c, pl.ds(row0, ROWS_PER_TILE)])


@functools.partial(jax.jit, static_argnames=())
def _sc_edge(z, s, d, m, src, dst, w):
    mesh = plsc.VectorSubcoreMesh(core_axis_name="c", subcore_axis_name="s")
    f = pl.kernel(
        _sc_edge_body,
        out_type=[
            jax.ShapeDtypeStruct((NC, NPAD, H), jnp.float32),
            jax.ShapeDtypeStruct((NC, NPAD), jnp.float32),
        ],
        mesh=mesh,
        compiler_params=pltpu.CompilerParams(needs_layout_passes=False,
                                             use_tc_tiling_on_sc=False),
        scratch_types=[
            pltpu.VMEM((NPAD,), jnp.float32),
            pltpu.VMEM((NPAD,), jnp.float32),
            pltpu.VMEM((16,), jnp.float32),
            pltpu.VMEM((CHUNKS, CHUNK), jnp.int32),
            pltpu.VMEM((CHUNKS, CHUNK), jnp.int32),
            pltpu.VMEM((CHUNKS, CHUNK), jnp.float32),
            pltpu.VMEM((CHUNK,), jnp.float32),
            pltpu.VMEM((CHUNK, H), jnp.float32),
            pltpu.VMEM((64, H), jnp.float32),
            pltpu.VMEM((ROWS_PER_TILE,), jnp.float32),
            pltpu.VMEM_SHARED((NPAD, H), jnp.float32),
            pltpu.VMEM_SHARED((NPAD,), jnp.float32),
            pltpu.SemaphoreType.DMA,
        ],
    )
    return f(z, s, d, m, src, dst, w)


def kernel(x, ei, ew, W_in, b_in, W1, b1, a1s, a1d, W2, b2, a2s, a2d,
           W_out, b_out):
    loop = jnp.arange(N, dtype=ei.dtype)
    pad = EPAD - ETOT
    src = jnp.concatenate([ei[0], loop, jnp.zeros((pad,), ei.dtype)])
    dst = jnp.concatenate([ei[1], loop, jnp.zeros((pad,), ei.dtype)])
    w = jnp.concatenate([ew, jnp.ones((N,), ew.dtype),
                         jnp.zeros((pad,), ew.dtype)])
    src = src.reshape(NW, CHUNKS, CHUNK).astype(jnp.int32)
    dst = dst.reshape(NW, CHUNKS, CHUNK).astype(jnp.int32)
    w = w.reshape(NW, CHUNKS, CHUNK)

    xp = jnp.pad(x, ((0, NPAD - N), (0, 0)))
    x1 = _inproj(xp, W_in, b_in)

    z1, s1, d1, m1 = _proj(x1, W1, b1, a1s, a1d)
    m1v = jnp.broadcast_to(m1.reshape(()), (16,))
    num1, den1 = _sc_edge(z1, s1.reshape(NPAD), d1.reshape(NPAD), m1v,
                          src, dst, w)
    x2 = _combine(num1, den1)

    z2, s2, d2, m2 = _proj(x2, W2, b2, a2s, a2d)
    m2v = jnp.broadcast_to(m2.reshape(()), (16,))
    num2, den2 = _sc_edge(z2, s2.reshape(NPAD), d2.reshape(NPAD), m2v,
                          src, dst, w)
    x3 = _combine(num2, den2)

    y = _outproj(x1, x2, x3, W_out, b_out)
    return y[:N]
